# baseline (device time: 50962 ns/iter reference)
import jax
import jax.numpy as jnp
from jax import lax
from jax.experimental import pallas as pl
from jax.experimental.pallas import tpu as pltpu

N_DEV = 4
BF16 = jnp.bfloat16
F32 = jnp.float32


def kernel(dy, W):
    m, k = dy.shape
    d, _ = W.shape
    mh = m // 2
    mq = m // 4
    dh = d // 2

    def body(dy_ref, w_ref, out_ref,
             s1a, s1b, c1a, c1b,
             s2a, s2b, c2a, c2b,
             s3a, s3b, r3a, r3b,
             r4a, r4b,
             send_sems, recv_sems):
        p = lax.axis_index("i")
        x = p // 2
        y = (p // 2) ^ (p % 2)
        px = 3 - p
        py = p ^ 1

        barrier_sem = pltpu.get_barrier_semaphore()
        for nbr in [px, py]:
            pl.semaphore_signal(
                barrier_sem, inc=1,
                device_id=(nbr,), device_id_type=pl.DeviceIdType.MESH,
            )
        pl.semaphore_wait(barrier_sem, 2)

        cols_a = pl.ds(0, dh)
        cols_b = pl.ds(dh, dh)

        def mk(idx, src, dst, partner):
            return pltpu.make_async_remote_copy(
                src_ref=src, dst_ref=dst,
                send_sem=send_sems.at[idx], recv_sem=recv_sems.at[idx],
                device_id=(partner,), device_id_type=pl.DeviceIdType.MESH,
            )

        def qdot(row0, wcol0):
            return lax.dot_general(
                dy_ref[pl.ds(row0, mq), :], w_ref[pl.ds(wcol0, dh), :],
                dimension_numbers=(((1,), (1,)), ((), ())),
                preferred_element_type=F32,
            )

        ra = x * mh + y * mq
        qa = x * mh + (1 - y) * mq
        rb = y * mh + x * mq
        qb = y * mh + (1 - x) * mq

        s1a[pl.ds((1 - y) * mq, mq), :] = qdot((1 - x) * mh + (1 - y) * mq, 0).astype(BF16)
        a1a = mk(0, s1a.at[pl.ds((1 - y) * mq, mq), :],
                 c1a.at[pl.ds((1 - y) * mq, mq), :], px)
        a1a.start()
        s1b[pl.ds((1 - x) * mq, mq), :] = qdot((1 - y) * mh + (1 - x) * mq, dh).astype(BF16)
        b1a = mk(1, s1b.at[pl.ds((1 - x) * mq, mq), :],
                 c1b.at[pl.ds((1 - x) * mq, mq), :], py)
        b1a.start()

        va = qdot(qa, 0)
        a1a.wait_recv()
        va = va + c1a[pl.ds((1 - y) * mq, mq), :].astype(F32)
        out_ref[pl.ds(qa, mq), cols_a] = va
        s2a[...] = va.astype(BF16)
        a2 = mk(2, s2a, c2a, py)
        a2.start()

        vb = qdot(qb, dh)
        b1a.wait_recv()
        vb = vb + c1b[pl.ds((1 - x) * mq, mq), :].astype(F32)
        out_ref[pl.ds(qb, mq), cols_b] = vb
        s2b[...] = vb.astype(BF16)
        b2 = mk(3, s2b, c2b, px)
        b2.start()

        s1a[pl.ds(y * mq, mq), :] = qdot((1 - x) * mh + y * mq, 0).astype(BF16)
        a1b = mk(4, s1a.at[pl.ds(y * mq, mq), :],
                 c1a.at[pl.ds(y * mq, mq), :], px)
        a1b.start()
        s1b[pl.ds(x * mq, mq), :] = qdot((1 - y) * mh + x * mq, dh).astype(BF16)
        b1b = mk(5, s1b.at[pl.ds(x * mq, mq), :],
                 c1b.at[pl.ds(x * mq, mq), :], py)
        b1b.start()

        wa = qdot(ra, 0)
        a1b.wait_recv()
        a2.wait_recv()
        wa = wa + (c1a[pl.ds(y * mq, mq), :].astype(F32) + c2a[...].astype(F32))
        out_ref[pl.ds(ra, mq), cols_a] = wa
        s3a[...] = wa.astype(BF16)
        a3 = mk(6, s3a, r3a, py)
        a3.start()
        a4i = mk(7, s3a, r4a.at[pl.ds(y * mq, mq), :], px)
        a4i.start()

        wb = qdot(rb, dh)
        b1b.wait_recv()
        b2.wait_recv()
        wb = wb + (c1b[pl.ds(x * mq, mq), :].astype(F32) + c2b[...].astype(F32))
        out_ref[pl.ds(rb, mq), cols_b] = wb
        s3b[...] = wb.astype(BF16)
        b3 = mk(8, s3b, r3b, px)
        b3.start()
        b4i = mk(9, s3b, r4b.at[pl.ds(x * mq, mq), :], py)
        b4i.start()

        a3.wait_recv()
        a4f = mk(10, r3a, r4a.at[pl.ds((1 - y) * mq, mq), :], px)
        a4f.start()
        out_ref[pl.ds(qa, mq), cols_a] = r3a[...].astype(F32)

        b3.wait_recv()
        b4f = mk(11, r3b, r4b.at[pl.ds((1 - x) * mq, mq), :], py)
        b4f.start()
        out_ref[pl.ds(qb, mq), cols_b] = r3b[...].astype(F32)

        a4i.wait_recv()
        a4f.wait_recv()
        out_ref[pl.ds((1 - x) * mh, mh), cols_a] = r4a[...].astype(F32)
        b4i.wait_recv()
        b4f.wait_recv()
        out_ref[pl.ds((1 - y) * mh, mh), cols_b] = r4b[...].astype(F32)

        for r in [a1a, b1a, a2, b2, a1b, b1b, a3, a4i, b3, b4i, a4f, b4f]:
            r.wait_send()

    return pl.pallas_call(
        body,
        out_shape=jax.ShapeDtypeStruct((m, d), F32),
        in_specs=[
            pl.BlockSpec(memory_space=pltpu.VMEM),
            pl.BlockSpec(memory_space=pltpu.VMEM),
        ],
        out_specs=pl.BlockSpec(memory_space=pltpu.VMEM),
        scratch_shapes=[
            pltpu.VMEM((mh, dh), BF16),
            pltpu.VMEM((mh, dh), BF16),
            pltpu.VMEM((mh, dh), BF16),
            pltpu.VMEM((mh, dh), BF16),
            pltpu.VMEM((mq, dh), BF16),
            pltpu.VMEM((mq, dh), BF16),
            pltpu.VMEM((mq, dh), BF16),
            pltpu.VMEM((mq, dh), BF16),
            pltpu.VMEM((mq, dh), BF16),
            pltpu.VMEM((mq, dh), BF16),
            pltpu.VMEM((mq, dh), BF16),
            pltpu.VMEM((mq, dh), BF16),
            pltpu.VMEM((mh, dh), BF16),
            pltpu.VMEM((mh, dh), BF16),
            pltpu.SemaphoreType.DMA((12,)),
            pltpu.SemaphoreType.DMA((12,)),
        ],
        compiler_params=pltpu.CompilerParams(
            collective_id=0, vmem_limit_bytes=100 * 1024 * 1024,
        ),
    )(dy, W)


# device time: 47117 ns/iter; 1.0816x vs baseline; 1.0816x over previous
import jax
import jax.numpy as jnp
from jax import lax
from jax.experimental import pallas as pl
from jax.experimental.pallas import tpu as pltpu

N_DEV = 4
BF16 = jnp.bfloat16
F32 = jnp.float32


def kernel(dy, W):
    m, k = dy.shape
    d, _ = W.shape
    mh = m // 2
    mq = m // 4
    dh = d // 2

    def body(dy_ref, w_ref, out_ref,
             s1a, s1b, c1a, c1b,
             s2a, s2b, c2a, c2b,
             s3a, s3b, r3a, r3b,
             r4a, r4b,
             send_sems, recv_sems):
        p = lax.axis_index("i")
        x = p // 2
        y = (p // 2) ^ (p % 2)
        px = 3 - p
        py = p ^ 1

        barrier_sem = pltpu.get_barrier_semaphore()
        for nbr in [px, py]:
            pl.semaphore_signal(
                barrier_sem, inc=1,
                device_id=(nbr,), device_id_type=pl.DeviceIdType.MESH,
            )
        pl.semaphore_wait(barrier_sem, 2)

        cols_a = pl.ds(0, dh)
        cols_b = pl.ds(dh, dh)

        def mk(idx, src, dst, partner):
            return pltpu.make_async_remote_copy(
                src_ref=src, dst_ref=dst,
                send_sem=send_sems.at[idx], recv_sem=recv_sems.at[idx],
                device_id=(partner,), device_id_type=pl.DeviceIdType.MESH,
            )

        def qdot(row0, wcol0):
            return lax.dot_general(
                dy_ref[pl.ds(row0, mq), :], w_ref[pl.ds(wcol0, dh), :],
                dimension_numbers=(((1,), (1,)), ((), ())),
                preferred_element_type=F32,
            )

        ra = x * mh + y * mq
        qa = x * mh + (1 - y) * mq
        rb = y * mh + x * mq
        qb = y * mh + (1 - x) * mq

        s1a[pl.ds((1 - y) * mq, mq), :] = qdot((1 - x) * mh + (1 - y) * mq, 0).astype(BF16)
        a1a = mk(0, s1a.at[pl.ds((1 - y) * mq, mq), :],
                 c1a.at[pl.ds((1 - y) * mq, mq), :], px)
        a1a.start()
        s1b[pl.ds((1 - x) * mq, mq), :] = qdot((1 - y) * mh + (1 - x) * mq, dh).astype(BF16)
        b1a = mk(1, s1b.at[pl.ds((1 - x) * mq, mq), :],
                 c1b.at[pl.ds((1 - x) * mq, mq), :], py)
        b1a.start()

        out_ref[pl.ds(qa, mq), cols_a] = qdot(qa, 0)
        a1a.wait_recv()
        out_ref[pl.ds(qa, mq), cols_a] += c1a[pl.ds((1 - y) * mq, mq), :].astype(F32)
        s2a[...] = out_ref[pl.ds(qa, mq), cols_a].astype(BF16)
        a2 = mk(2, s2a, c2a, py)
        a2.start()

        out_ref[pl.ds(qb, mq), cols_b] = qdot(qb, dh)
        b1a.wait_recv()
        out_ref[pl.ds(qb, mq), cols_b] += c1b[pl.ds((1 - x) * mq, mq), :].astype(F32)
        s2b[...] = out_ref[pl.ds(qb, mq), cols_b].astype(BF16)
        b2 = mk(3, s2b, c2b, px)
        b2.start()

        s1a[pl.ds(y * mq, mq), :] = qdot((1 - x) * mh + y * mq, 0).astype(BF16)
        a1b = mk(4, s1a.at[pl.ds(y * mq, mq), :],
                 c1a.at[pl.ds(y * mq, mq), :], px)
        a1b.start()
        s1b[pl.ds(x * mq, mq), :] = qdot((1 - y) * mh + x * mq, dh).astype(BF16)
        b1b = mk(5, s1b.at[pl.ds(x * mq, mq), :],
                 c1b.at[pl.ds(x * mq, mq), :], py)
        b1b.start()

        out_ref[pl.ds(ra, mq), cols_a] = qdot(ra, 0)
        a1b.wait_recv()
        out_ref[pl.ds(ra, mq), cols_a] += c1a[pl.ds(y * mq, mq), :].astype(F32)
        a2.wait_recv()
        out_ref[pl.ds(ra, mq), cols_a] += c2a[...].astype(F32)
        s3a[...] = out_ref[pl.ds(ra, mq), cols_a].astype(BF16)
        a3 = mk(6, s3a, r3a, py)
        a3.start()
        a4i = mk(7, s3a, r4a.at[pl.ds(y * mq, mq), :], px)
        a4i.start()

        out_ref[pl.ds(rb, mq), cols_b] = qdot(rb, dh)
        b1b.wait_recv()
        out_ref[pl.ds(rb, mq), cols_b] += c1b[pl.ds(x * mq, mq), :].astype(F32)
        b2.wait_recv()
        out_ref[pl.ds(rb, mq), cols_b] += c2b[...].astype(F32)
        s3b[...] = out_ref[pl.ds(rb, mq), cols_b].astype(BF16)
        b3 = mk(8, s3b, r3b, px)
        b3.start()
        b4i = mk(9, s3b, r4b.at[pl.ds(x * mq, mq), :], py)
        b4i.start()

        a3.wait_recv()
        a4f = mk(10, r3a, r4a.at[pl.ds((1 - y) * mq, mq), :], px)
        a4f.start()
        out_ref[pl.ds(qa, mq), cols_a] = r3a[...].astype(F32)

        b3.wait_recv()
        b4f = mk(11, r3b, r4b.at[pl.ds((1 - x) * mq, mq), :], py)
        b4f.start()
        out_ref[pl.ds(qb, mq), cols_b] = r3b[...].astype(F32)

        a4i.wait_recv()
        a4f.wait_recv()
        out_ref[pl.ds((1 - x) * mh, mh), cols_a] = r4a[...].astype(F32)
        b4i.wait_recv()
        b4f.wait_recv()
        out_ref[pl.ds((1 - y) * mh, mh), cols_b] = r4b[...].astype(F32)

        for r in [a1a, b1a, a2, b2, a1b, b1b, a3, a4i, b3, b4i, a4f, b4f]:
            r.wait_send()

    return pl.pallas_call(
        body,
        out_shape=jax.ShapeDtypeStruct((m, d), F32),
        in_specs=[
            pl.BlockSpec(memory_space=pltpu.VMEM),
            pl.BlockSpec(memory_space=pltpu.VMEM),
        ],
        out_specs=pl.BlockSpec(memory_space=pltpu.VMEM),
        scratch_shapes=[
            pltpu.VMEM((mh, dh), BF16),
            pltpu.VMEM((mh, dh), BF16),
            pltpu.VMEM((mh, dh), BF16),
            pltpu.VMEM((mh, dh), BF16),
            pltpu.VMEM((mq, dh), BF16),
            pltpu.VMEM((mq, dh), BF16),
            pltpu.VMEM((mq, dh), BF16),
            pltpu.VMEM((mq, dh), BF16),
            pltpu.VMEM((mq, dh), BF16),
            pltpu.VMEM((mq, dh), BF16),
            pltpu.VMEM((mq, dh), BF16),
            pltpu.VMEM((mq, dh), BF16),
            pltpu.VMEM((mh, dh), BF16),
            pltpu.VMEM((mh, dh), BF16),
            pltpu.SemaphoreType.DMA((12,)),
            pltpu.SemaphoreType.DMA((12,)),
        ],
        compiler_params=pltpu.CompilerParams(
            collective_id=0, vmem_limit_bytes=100 * 1024 * 1024,
        ),
    )(dy, W)
